# Initial kernel scaffold; baseline (speedup 1.0000x reference)
import jax
import jax.numpy as jnp
from jax.experimental import pallas as pl


def kernel(x, emb_table, temporal_table):
    raise NotImplementedError("write your pallas kernel here")


# speedup vs baseline: 19.1170x; 19.1170x over previous
import functools

import jax
import jax.numpy as jnp
from jax import lax
from jax.experimental import pallas as pl
from jax.experimental.pallas import tpu as pltpu
from jax.experimental.pallas import tpu_sc as plsc

_VOCAB, _SENT, _MEM, _EMB, _BATCH = 100000, 20, 50, 32, 1024
_NW = 32
_NSEG = _BATCH * _MEM
_SPW = _NSEG // _NW
_BLK = 40
_NBLK = _SPW // _BLK
_IDXPD = 100
_DPB = _BLK * _SENT // _IDXPD
_NROW = _SPW * _SENT // _IDXPD

_SCALE = 4.0 / (_EMB * _SENT)


def _u(s):
    return float((s + 1 - (_SENT + 1) / 2.0) * _SCALE)


_mesh = plsc.VectorSubcoreMesh(core_axis_name="c", subcore_axis_name="s")


@functools.partial(
    pl.kernel,
    mesh=_mesh,
    compiler_params=pltpu.CompilerParams(use_tc_tiling_on_sc=False),
    out_type=jax.ShapeDtypeStruct((_NSEG, _EMB), jnp.float32),
    scratch_types=[
        pltpu.VMEM((_NROW, _IDXPD), jnp.int32),
        pltpu.VMEM((_BLK * _SENT, _EMB), jnp.float32),
        pltpu.VMEM((_BLK, _EMB), jnp.float32),
        pltpu.VMEM((_MEM, _EMB), jnp.float32),
        pltpu.SemaphoreType.DMA,
    ],
)
def _emb_kernel(x_hbm, tab_hbm, temp_hbm, out_hbm,
                idx_v, rows_v, outb_v, temp_v, sem):
    wid = lax.axis_index("s") * 2 + lax.axis_index("c")
    pltpu.sync_copy(x_hbm.at[wid], idx_v)
    pltpu.sync_copy(temp_hbm, temp_v)

    v0 = lax.iota(jnp.int32, 16).astype(jnp.float32) - 15.5
    v1 = v0 + 16.0
    base_seg = wid * _SPW

    def block(blk, carry):
        cps = [
            pltpu.async_copy(
                tab_hbm.at[idx_v.at[blk * _DPB + d]],
                rows_v.at[pl.ds(d * _IDXPD, _IDXPD)],
                sem,
            )
            for d in range(_DPB)
        ]
        for cp in cps:
            cp.wait()

        def seg(i, c2):
            m = lax.rem(blk * _BLK + i, _MEM)
            r = i * _SENT
            acc0 = rows_v[r, pl.ds(0, 16)] * _u(0)
            acc1 = rows_v[r, pl.ds(16, 16)] * _u(0)
            for s in range(1, _SENT - 1):
                acc0 = acc0 + rows_v[r + s, pl.ds(0, 16)] * _u(s)
                acc1 = acc1 + rows_v[r + s, pl.ds(16, 16)] * _u(s)
            last0 = rows_v[r + _SENT - 1, pl.ds(0, 16)] + temp_v[m, pl.ds(0, 16)]
            last1 = rows_v[r + _SENT - 1, pl.ds(16, 16)] + temp_v[m, pl.ds(16, 16)]
            outb_v[i, pl.ds(0, 16)] = acc0 * v0 + last0
            outb_v[i, pl.ds(16, 16)] = acc1 * v1 + last1
            return c2

        lax.fori_loop(0, _BLK, seg, 0)
        pltpu.sync_copy(outb_v, out_hbm.at[pl.ds(base_seg + blk * _BLK, _BLK)])
        return carry

    lax.fori_loop(0, _NBLK, block, 0)


def kernel(x, emb_table, temporal_table):
    xi = x.astype(jnp.int32).reshape(_NW, _NROW, _IDXPD)
    out = _emb_kernel(xi, emb_table, temporal_table)
    return out.reshape(_BATCH, _MEM, _EMB)


# speedup vs baseline: 24.3112x; 1.2717x over previous
import functools

import jax
import jax.numpy as jnp
from jax import lax
from jax.experimental import pallas as pl
from jax.experimental.pallas import tpu as pltpu
from jax.experimental.pallas import tpu_sc as plsc

_VOCAB, _SENT, _MEM, _EMB, _BATCH = 100000, 20, 50, 32, 1024
_NW = 32
_NSEG = _BATCH * _MEM
_SPW = _NSEG // _NW
_BLK = 32
_NBLK = _SPW // _BLK
_NPAIR = _NBLK // 2
_IDXPD = 128
_DPB = _BLK * _SENT // _IDXPD
_NROW = _SPW * _SENT // _IDXPD

_SCALE = 4.0 / (_EMB * _SENT)


def _u(s):
    return float((s + 1 - (_SENT + 1) / 2.0) * _SCALE)


_mesh = plsc.VectorSubcoreMesh(core_axis_name="c", subcore_axis_name="s")


@functools.partial(
    pl.kernel,
    mesh=_mesh,
    compiler_params=pltpu.CompilerParams(use_tc_tiling_on_sc=False),
    out_type=jax.ShapeDtypeStruct((_NSEG, _EMB), jnp.float32),
    scratch_types=[
        pltpu.VMEM((_NROW, _IDXPD), jnp.int32),
        pltpu.VMEM((_BLK * _SENT, _EMB), jnp.float32),
        pltpu.VMEM((_BLK * _SENT, _EMB), jnp.float32),
        pltpu.VMEM((_BLK, _EMB), jnp.float32),
        pltpu.VMEM((_BLK, _EMB), jnp.float32),
        pltpu.VMEM((_MEM, _EMB), jnp.float32),
        pltpu.SemaphoreType.DMA,
        pltpu.SemaphoreType.DMA,
        pltpu.SemaphoreType.DMA,
        pltpu.SemaphoreType.DMA,
    ],
)
def _emb_kernel(x_hbm, tab_hbm, temp_hbm, out_hbm,
                idx_v, rows0, rows1, outb0, outb1, temp_v,
                gsem0, gsem1, osem0, osem1):
    wid = lax.axis_index("s") * 2 + lax.axis_index("c")
    pltpu.sync_copy(x_hbm.at[wid], idx_v)
    pltpu.sync_copy(temp_hbm, temp_v)

    v0 = lax.iota(jnp.int32, 16).astype(jnp.float32) - 15.5
    v1 = v0 + 16.0
    base_seg = wid * _SPW

    def fire(blk, rows_buf, gsem):
        for d in range(_DPB):
            pltpu.async_copy(
                tab_hbm.at[idx_v.at[blk * _DPB + d]],
                rows_buf.at[pl.ds(d * _IDXPD, _IDXPD)],
                gsem,
            )

    def wait_gather(rows_buf, gsem):
        pltpu.make_async_copy(
            tab_hbm.at[pl.ds(0, _BLK * _SENT)], rows_buf, gsem).wait()

    def fire_out(blk, outb, osem):
        pltpu.async_copy(
            outb, out_hbm.at[pl.ds(base_seg + blk * _BLK, _BLK)], osem)

    def wait_out(outb, osem):
        pltpu.make_async_copy(
            outb, out_hbm.at[pl.ds(base_seg, _BLK)], osem).wait()

    def compute(blk, rows_buf, outb):
        def seg(i, c2):
            m = lax.rem(blk * _BLK + i, _MEM)
            r = i * _SENT
            acc0 = rows_buf[r, pl.ds(0, 16)] * _u(0)
            acc1 = rows_buf[r, pl.ds(16, 16)] * _u(0)
            for s in range(1, _SENT - 1):
                acc0 = acc0 + rows_buf[r + s, pl.ds(0, 16)] * _u(s)
                acc1 = acc1 + rows_buf[r + s, pl.ds(16, 16)] * _u(s)
            last0 = rows_buf[r + _SENT - 1, pl.ds(0, 16)] + temp_v[m, pl.ds(0, 16)]
            last1 = rows_buf[r + _SENT - 1, pl.ds(16, 16)] + temp_v[m, pl.ds(16, 16)]
            outb[i, pl.ds(0, 16)] = acc0 * v0 + last0
            outb[i, pl.ds(16, 16)] = acc1 * v1 + last1
            return c2

        lax.fori_loop(0, _BLK, seg, 0)

    fire(0, rows0, gsem0)

    def pair(p, carry):
        ga = 2 * p
        gb = ga + 1
        fire(gb, rows1, gsem1)
        wait_gather(rows0, gsem0)

        @pl.when(p >= 1)
        def _():
            wait_out(outb0, osem0)

        compute(ga, rows0, outb0)
        fire_out(ga, outb0, osem0)

        @pl.when(p <= _NPAIR - 2)
        def _():
            fire(gb + 1, rows0, gsem0)

        wait_gather(rows1, gsem1)

        @pl.when(p >= 1)
        def _():
            wait_out(outb1, osem1)

        compute(gb, rows1, outb1)
        fire_out(gb, outb1, osem1)
        return carry

    lax.fori_loop(0, _NPAIR, pair, 0)
    wait_out(outb0, osem0)
    wait_out(outb1, osem1)


def kernel(x, emb_table, temporal_table):
    xi = x.astype(jnp.int32).reshape(_NW, _NROW, _IDXPD)
    out = _emb_kernel(xi, emb_table, temporal_table)
    return out.reshape(_BATCH, _MEM, _EMB)


# speedup vs baseline: 25.6315x; 1.0543x over previous
import functools

import jax
import jax.numpy as jnp
from jax import lax
from jax.experimental import pallas as pl
from jax.experimental.pallas import tpu as pltpu
from jax.experimental.pallas import tpu_sc as plsc

_VOCAB, _SENT, _MEM, _EMB, _BATCH = 100000, 20, 50, 32, 1024
_NW = 32
_BPW = _BATCH // _NW
_NPAIR = _BPW // 2

_SCALE = 4.0 / (_EMB * _SENT)


def _u(s):
    return float((s + 1 - (_SENT + 1) / 2.0) * _SCALE)


_mesh = plsc.VectorSubcoreMesh(core_axis_name="c", subcore_axis_name="s")


@functools.partial(
    pl.kernel,
    mesh=_mesh,
    compiler_params=pltpu.CompilerParams(use_tc_tiling_on_sc=False),
    out_type=jax.ShapeDtypeStruct((_BATCH, _MEM, _EMB), jnp.float32),
    scratch_types=[
        pltpu.VMEM((_BPW, _MEM, _SENT), jnp.int32),
        pltpu.VMEM((_MEM * _SENT, _EMB), jnp.float32),
        pltpu.VMEM((_MEM * _SENT, _EMB), jnp.float32),
        pltpu.VMEM((_MEM, _EMB), jnp.float32),
        pltpu.VMEM((_MEM, _EMB), jnp.float32),
        pltpu.VMEM((_MEM, _EMB), jnp.float32),
        pltpu.SemaphoreType.DMA,
        pltpu.SemaphoreType.DMA,
        pltpu.SemaphoreType.DMA,
        pltpu.SemaphoreType.DMA,
    ],
)
def _emb_kernel(x_hbm, tab_hbm, temp_hbm, out_hbm,
                idx_v, rows0, rows1, outb0, outb1, temp_v,
                gsem0, gsem1, osem0, osem1):
    wid = lax.axis_index("s") * 2 + lax.axis_index("c")
    base_b = wid * _BPW
    pltpu.sync_copy(x_hbm.at[pl.ds(base_b, _BPW)], idx_v)
    pltpu.sync_copy(temp_hbm, temp_v)

    v0 = lax.iota(jnp.int32, 16).astype(jnp.float32) - 15.5
    v1 = v0 + 16.0

    def fire(b, rows_buf, gsem):
        for m in range(_MEM):
            pltpu.async_copy(
                tab_hbm.at[idx_v.at[b, m]],
                rows_buf.at[pl.ds(m * _SENT, _SENT)],
                gsem,
            )

    def wait_gather(rows_buf, gsem):
        pltpu.make_async_copy(
            tab_hbm.at[pl.ds(0, _MEM * _SENT)], rows_buf, gsem).wait()

    def fire_out(b, outb, osem):
        pltpu.async_copy(outb, out_hbm.at[base_b + b], osem)

    def wait_out(outb, osem):
        pltpu.make_async_copy(outb, out_hbm.at[base_b], osem).wait()

    def compute(rows_buf, outb):
        def seg(i, c2):
            r = i * _SENT
            acc0 = rows_buf[r, pl.ds(0, 16)] * _u(0)
            acc1 = rows_buf[r, pl.ds(16, 16)] * _u(0)
            for s in range(1, _SENT - 1):
                acc0 = acc0 + rows_buf[r + s, pl.ds(0, 16)] * _u(s)
                acc1 = acc1 + rows_buf[r + s, pl.ds(16, 16)] * _u(s)
            last0 = rows_buf[r + _SENT - 1, pl.ds(0, 16)] + temp_v[i, pl.ds(0, 16)]
            last1 = rows_buf[r + _SENT - 1, pl.ds(16, 16)] + temp_v[i, pl.ds(16, 16)]
            outb[i, pl.ds(0, 16)] = acc0 * v0 + last0
            outb[i, pl.ds(16, 16)] = acc1 * v1 + last1
            return c2

        lax.fori_loop(0, _MEM, seg, 0)

    fire(0, rows0, gsem0)

    def pair(p, carry):
        ba = 2 * p
        bb = ba + 1
        fire(bb, rows1, gsem1)
        wait_gather(rows0, gsem0)

        @pl.when(p >= 1)
        def _():
            wait_out(outb0, osem0)

        compute(rows0, outb0)
        fire_out(ba, outb0, osem0)

        @pl.when(p <= _NPAIR - 2)
        def _():
            fire(bb + 1, rows0, gsem0)

        wait_gather(rows1, gsem1)

        @pl.when(p >= 1)
        def _():
            wait_out(outb1, osem1)

        compute(rows1, outb1)
        fire_out(bb, outb1, osem1)
        return carry

    lax.fori_loop(0, _NPAIR, pair, 0)
    wait_out(outb0, osem0)
    wait_out(outb1, osem1)


def kernel(x, emb_table, temporal_table):
    return _emb_kernel(x.astype(jnp.int32), emb_table, temporal_table)


# speedup vs baseline: 29.0377x; 1.1329x over previous
import functools

import jax
import jax.numpy as jnp
from jax import lax
from jax.experimental import pallas as pl
from jax.experimental.pallas import tpu as pltpu
from jax.experimental.pallas import tpu_sc as plsc

_VOCAB, _SENT, _MEM, _EMB, _BATCH = 100000, 20, 50, 32, 1024
_NW = 32
_BPW = _BATCH // _NW
_NPAIR = _BPW // 2
_IPB = _MEM * _SENT
_IPW = _BPW * _IPB
_IDXPD = 40
_DPB = _IPB // _IDXPD

_SCALE = 4.0 / (_EMB * _SENT)


def _u(s):
    return float((s + 1 - (_SENT + 1) / 2.0) * _SCALE)


_mesh = plsc.VectorSubcoreMesh(core_axis_name="c", subcore_axis_name="s")


@functools.partial(
    pl.kernel,
    mesh=_mesh,
    compiler_params=pltpu.CompilerParams(use_tc_tiling_on_sc=False),
    out_type=jax.ShapeDtypeStruct((_BATCH, _MEM, _EMB), jnp.float32),
    scratch_types=[
        pltpu.VMEM((_IPW,), jnp.int32),
        pltpu.VMEM((_IPB, _EMB), jnp.float32),
        pltpu.VMEM((_IPB, _EMB), jnp.float32),
        pltpu.VMEM((_MEM, _EMB), jnp.float32),
        pltpu.VMEM((_MEM, _EMB), jnp.float32),
        pltpu.VMEM((_MEM, _EMB), jnp.float32),
        pltpu.SemaphoreType.DMA,
        pltpu.SemaphoreType.DMA,
        pltpu.SemaphoreType.DMA,
        pltpu.SemaphoreType.DMA,
    ],
)
def _emb_kernel(x_hbm, tab_hbm, temp_hbm, out_hbm,
                idx_v, rows0, rows1, outb0, outb1, temp_v,
                gsem0, gsem1, osem0, osem1):
    wid = lax.axis_index("s") * 2 + lax.axis_index("c")
    base_b = wid * _BPW
    pltpu.sync_copy(x_hbm.at[pl.ds(wid * _IPW, _IPW)], idx_v)
    pltpu.sync_copy(temp_hbm, temp_v)

    v0 = lax.iota(jnp.int32, 16).astype(jnp.float32) - 15.5
    v1 = v0 + 16.0

    def fire(b, rows_buf, gsem):
        base = b * _IPB
        for j in range(_DPB):
            pltpu.async_copy(
                tab_hbm.at[idx_v.at[pl.ds(base + j * _IDXPD, _IDXPD)]],
                rows_buf.at[pl.ds(j * _IDXPD, _IDXPD)],
                gsem,
            )

    def wait_gather(rows_buf, gsem):
        pltpu.make_async_copy(
            tab_hbm.at[pl.ds(0, _IPB)], rows_buf, gsem).wait()

    def fire_out(b, outb, osem):
        pltpu.async_copy(outb, out_hbm.at[base_b + b], osem)

    def wait_out(outb, osem):
        pltpu.make_async_copy(outb, out_hbm.at[base_b], osem).wait()

    def one_seg(rows_buf, outb, i):
        r = i * _SENT
        acc0 = rows_buf[r, pl.ds(0, 16)] * _u(0)
        acc1 = rows_buf[r, pl.ds(16, 16)] * _u(0)
        for s in range(1, _SENT - 1):
            acc0 = acc0 + rows_buf[r + s, pl.ds(0, 16)] * _u(s)
            acc1 = acc1 + rows_buf[r + s, pl.ds(16, 16)] * _u(s)
        last0 = rows_buf[r + _SENT - 1, pl.ds(0, 16)] + temp_v[i, pl.ds(0, 16)]
        last1 = rows_buf[r + _SENT - 1, pl.ds(16, 16)] + temp_v[i, pl.ds(16, 16)]
        outb[i, pl.ds(0, 16)] = acc0 * v0 + last0
        outb[i, pl.ds(16, 16)] = acc1 * v1 + last1

    def compute(rows_buf, outb):
        def seg2(k, c2):
            i = k * 2
            one_seg(rows_buf, outb, i)
            one_seg(rows_buf, outb, i + 1)
            return c2

        lax.fori_loop(0, _MEM // 2, seg2, 0)

    fire(0, rows0, gsem0)

    def pair(p, carry):
        ba = 2 * p
        bb = ba + 1
        fire(bb, rows1, gsem1)
        wait_gather(rows0, gsem0)

        @pl.when(p >= 1)
        def _():
            wait_out(outb0, osem0)

        compute(rows0, outb0)
        fire_out(ba, outb0, osem0)

        @pl.when(p <= _NPAIR - 2)
        def _():
            fire(bb + 1, rows0, gsem0)

        wait_gather(rows1, gsem1)

        @pl.when(p >= 1)
        def _():
            wait_out(outb1, osem1)

        compute(rows1, outb1)
        fire_out(bb, outb1, osem1)
        return carry

    lax.fori_loop(0, _NPAIR, pair, 0)
    wait_out(outb0, osem0)
    wait_out(outb1, osem1)


def kernel(x, emb_table, temporal_table):
    return _emb_kernel(x.astype(jnp.int32).reshape(-1), emb_table,
                       temporal_table)
